# baseline (speedup 1.0000x reference)
import functools

import jax
import jax.numpy as jnp
from jax import lax
from jax.experimental import pallas as pl
from jax.experimental.pallas import tpu as pltpu
from jax.experimental.pallas import tpu_sc as plsc

D = 128
CB = 128
NW = 32
N_SUB = 16


G = 8


def _sc_aggregate(h, zeros_pad, src_t, dst_t, ch, npad):
    ngroups = ch // G
    rows_per_tile = npad // N_SUB
    mesh = plsc.VectorSubcoreMesh(core_axis_name="c", subcore_axis_name="s")

    @functools.partial(
        pl.kernel,
        out_type=jax.ShapeDtypeStruct((2, npad, D), jnp.float32),
        mesh=mesh,
        scratch_types=[
            pltpu.VMEM((2, G, CB), jnp.int32),
            pltpu.VMEM((2, G, CB), jnp.int32),
            pltpu.VMEM((2, CB, D), jnp.float32),
            pltpu.VMEM_SHARED((npad, D), jnp.float32),
            pltpu.SemaphoreType.DMA,
            pltpu.SemaphoreType.DMA,
            pltpu.SemaphoreType.DMA,
        ],
    )
    def agg(h_hbm, z_hbm, src_hbm, dst_hbm, out_hbm,
            src_v, dst_v, rows_v, acc, gsem, ssem, isem):
        cid = lax.axis_index("c")
        sid = lax.axis_index("s")
        wid = cid * N_SUB + sid
        r0 = sid * rows_per_tile
        pltpu.sync_copy(z_hbm.at[pl.ds(r0, rows_per_tile)],
                        acc.at[pl.ds(r0, rows_per_tile)])
        pltpu.sync_copy(src_hbm.at[wid, 0], src_v.at[0])
        pltpu.sync_copy(dst_hbm.at[wid, 0], dst_v.at[0])
        plsc.subcore_barrier()

        def gather(ip, c, b):
            return pltpu.make_async_copy(
                h_hbm.at[src_v.at[ip, c]], rows_v.at[b], gsem)

        def scatter(ip, c, b):
            return pltpu.make_async_copy(
                rows_v.at[b], acc.at[dst_v.at[ip, c]], ssem)

        def idx_load(g, ip):
            return (pltpu.make_async_copy(src_hbm.at[wid, g], src_v.at[ip],
                                          isem),
                    pltpu.make_async_copy(dst_hbm.at[wid, g], dst_v.at[ip],
                                          isem))

        gather(0, 0, 0).start()

        def group(g, ip):
            @pl.when(g + 1 < ngroups)
            def _():
                for cp in idx_load(g + 1, 1 - ip):
                    cp.start()

            for c in range(G):
                b = c % 2
                gather(ip, c, b).wait()
                scatter(ip, c, b).start(add=True)
                if c == 0:
                    @pl.when(g >= 1)
                    def _():
                        scatter(ip, c, 1 - b).wait()
                else:
                    scatter(ip, c, 1 - b).wait()
                if c < G - 1:
                    gather(ip, c + 1, 1 - b).start()
                else:
                    @pl.when(g + 1 < ngroups)
                    def _():
                        for cp in idx_load(g + 1, 1 - ip):
                            cp.wait()
                        gather(1 - ip, 0, 1 - b).start()

        def body(i, carry):
            group(2 * i, 0)
            group(2 * i + 1, 1)
            return carry

        lax.fori_loop(0, ngroups // 2, body, 0)
        scatter(1, G - 1, (ch - 1) % 2).wait()
        plsc.subcore_barrier()
        pltpu.sync_copy(acc.at[pl.ds(r0, rows_per_tile)],
                        out_hbm.at[cid, pl.ds(r0, rows_per_tile)])

    return agg(h, zeros_pad, src_t, dst_t)


def _mlp_call(partials, h, Wa, ba, Wb, bb, final_relu):
    n = h.shape[0]
    br = 1000
    grid = (n // br,)

    def body(p_ref, h_ref, wa_ref, ba_ref, wb_ref, bb_ref, o_ref):
        a = h_ref[...] + p_ref[0] + p_ref[1]
        t = jnp.dot(a, wa_ref[...], preferred_element_type=jnp.float32)
        t = jnp.maximum(t + ba_ref[...], 0.0)
        t = jnp.dot(t, wb_ref[...], preferred_element_type=jnp.float32)
        t = t + bb_ref[...]
        if final_relu:
            t = jnp.maximum(t, 0.0)
        o_ref[...] = t

    return pl.pallas_call(
        body,
        grid=grid,
        in_specs=[
            pl.BlockSpec((2, br, D), lambda i: (0, i, 0)),
            pl.BlockSpec((br, D), lambda i: (i, 0)),
            pl.BlockSpec((D, D), lambda i: (0, 0)),
            pl.BlockSpec((1, D), lambda i: (0, 0)),
            pl.BlockSpec((D, D), lambda i: (0, 0)),
            pl.BlockSpec((1, D), lambda i: (0, 0)),
        ],
        out_specs=pl.BlockSpec((br, D), lambda i: (i, 0)),
        out_shape=jax.ShapeDtypeStruct((n, D), jnp.float32),
    )(partials, h, Wa, ba.reshape(1, D), Wb, bb.reshape(1, D))


def kernel(x, edge_index, W1a, b1a, W1b, b1b, W2a, b2a, W2b, b2b):
    n = x.shape[0]
    npad = ((n + 127) // 128) * 128 + 128 if n % 128 == 0 else -(-n // 128) * 128
    src = edge_index[0].astype(jnp.int32)
    dst = edge_index[1].astype(jnp.int32)
    e = src.shape[0]
    per_tile = -(-e // NW)
    ch = -(-per_tile // CB)
    ch = -(-ch // (2 * G)) * (2 * G)
    e_pad = NW * ch * CB
    src_p = jnp.concatenate(
        [src, jnp.zeros((e_pad - e,), jnp.int32)]).reshape(NW, ch // G, G, CB)
    dst_p = jnp.concatenate(
        [dst, jnp.full((e_pad - e,), n, jnp.int32)]).reshape(NW, ch // G, G, CB)
    zeros_pad = jnp.zeros((npad, D), jnp.float32)

    p1 = _sc_aggregate(x, zeros_pad, src_p, dst_p, ch, npad)
    h1 = _mlp_call(p1, x, W1a, b1a, W1b, b1b, final_relu=True)
    p2 = _sc_aggregate(h1, zeros_pad, src_p, dst_p, ch, npad)
    out = _mlp_call(p2, h1, W2a, b2a, W2b, b2b, final_relu=False)
    return out
